# baseline (speedup 1.0000x reference)
import jax
import jax.numpy as jnp
from jax import lax
from jax.experimental import pallas as pl
from jax.experimental.pallas import tpu as pltpu

_D_SPACE = 64
_N_QK, _N_V, _N_REL, _N_VAL = 256, 128, 64, 32
_K_QK, _K_V, _K_REL, _K_VAL = 64, 32, 16, 3
_ST = 1024


_N_PAD = 640
_REGIONS = ((0, 256, 0.0), (256, 128, 0.0), (384, 128, 64.0), (512, 128, 96.0))


def _fused_kernel(x_ref, imp_ref, wp_ref, bp_ref, ea_ref, eb_ref,
                  iqk_ref, iv_ref, rw_ref, vw_ref, acc_ref, en_ref):
    s = pl.program_id(1)
    ns = pl.num_programs(1)
    x = x_ref[0]
    h = jnp.dot(x, wp_ref[...], preferred_element_type=jnp.float32) + bp_ref[...]
    imp = imp_ref[0]

    @pl.when(s == 0)
    def _():
        e = jnp.concatenate([ea_ref[...], jnp.zeros((64, _D_SPACE), jnp.float32),
                             eb_ref[...], jnp.zeros((96, _D_SPACE), jnp.float32)],
                            axis=0)
        en_ref[...] = e / (jnp.sqrt(jnp.sum(e * e, axis=1, keepdims=True)) + 1e-12)

    lg = lax.dot_general(h, en_ref[...], (((1,), (1,)), ((), ())),
                         preferred_element_type=jnp.float32)
    ex = jnp.exp(lg)
    st = ex.shape[0]
    invs = []
    for lo, width, padn in _REGIONS:
        d = jnp.sum(ex[:, lo:lo + width], axis=1, keepdims=True) - padn
        invs.append(jnp.broadcast_to(1.0 / d, (st, width)))
    p = ex * jnp.concatenate(invs, axis=1)
    contrib = jnp.dot(imp, p, preferred_element_type=jnp.float32)

    @pl.when(s == 0)
    def _():
        acc_ref[...] = contrib

    @pl.when(s != 0)
    def _():
        acc_ref[...] += contrib

    @pl.when(s == ns - 1)
    def _():
        acc = acc_ref[...]
        for lo, n, k, i_ref in ((0, _N_QK, _K_QK, iqk_ref),
                                (256, _N_V, _K_V, iv_ref)):
            w_row = acc[:, lo:lo + n]
            sel = _row_sel(w_row, k).astype(jnp.float32)
            i_ref[0] = _row_sorted_idx(sel, k)
        for lo, n, k, o_ref in ((384, _N_REL, _K_REL, rw_ref),
                                (512, _N_VAL, _K_VAL, vw_ref)):
            w_row = acc[:, lo:lo + n]
            sw = w_row * _row_sel(w_row, k).astype(jnp.float32)
            o_ref[0] = sw / (jnp.sum(sw, axis=1, keepdims=True) + 1e-8)


def _col_bcast(row, m):
    ones = jnp.ones((1, m), jnp.float32)
    return lax.dot_general(row, ones, (((0,), (0,)), ((), ())),
                           precision=lax.Precision.HIGHEST,
                           preferred_element_type=jnp.float32)


def _row_sel(w_row, k):
    n = w_row.shape[1]
    wj = _col_bcast(w_row, n)
    wi = jnp.broadcast_to(w_row, (n, n))
    ij = lax.broadcasted_iota(jnp.int32, (n, n), 0)
    ii = lax.broadcasted_iota(jnp.int32, (n, n), 1)
    beats = (wj > wi) | ((wj == wi) & (ij < ii))
    rank = jnp.sum(beats.astype(jnp.float32), axis=0, keepdims=True)
    return rank < float(k)


def _row_sorted_idx(self, k):
    n = self.shape[1]
    ij = lax.broadcasted_iota(jnp.int32, (n, n), 0)
    ii = lax.broadcasted_iota(jnp.int32, (n, n), 1)
    tri = (ij <= ii).astype(jnp.float32)
    csum = jnp.dot(self, tri, precision=lax.Precision.HIGHEST,
                   preferred_element_type=jnp.float32)
    pos = csum - 1.0
    pos_mat = _col_bcast(pos, k)
    sel_mat = _col_bcast(self, k)
    kio = lax.broadcasted_iota(jnp.int32, (n, k), 1).astype(jnp.float32)
    iio = lax.broadcasted_iota(jnp.int32, (n, k), 0).astype(jnp.float32)
    onehot = sel_mat * (pos_mat == kio).astype(jnp.float32)
    idx = jnp.sum(onehot * iio, axis=0, keepdims=True)
    return idx.astype(jnp.int32)


def kernel(x, importance, W_proj, b_proj, neuron_emb):
    B, S, D = x.shape
    bp = b_proj.reshape(1, _D_SPACE)
    ns = S // _ST

    osizes = (_K_QK, _K_V, _N_REL, _N_VAL)
    imp3 = importance.reshape(B, 1, S)
    iqk, iv, rw, vw = pl.pallas_call(
        _fused_kernel,
        grid=(B, ns),
        in_specs=[
            pl.BlockSpec((1, _ST, D), lambda b, s: (b, s, 0)),
            pl.BlockSpec((1, 1, _ST), lambda b, s: (b, 0, s)),
            pl.BlockSpec((D, _D_SPACE), lambda b, s: (0, 0)),
            pl.BlockSpec((1, _D_SPACE), lambda b, s: (0, 0)),
            pl.BlockSpec((448, _D_SPACE), lambda b, s: (0, 0)),
            pl.BlockSpec((32, _D_SPACE), lambda b, s: (14, 0)),
        ],
        out_specs=[pl.BlockSpec((1, 1, n), lambda b, s: (b, 0, 0)) for n in osizes],
        out_shape=[
            jax.ShapeDtypeStruct((B, 1, _K_QK), jnp.int32),
            jax.ShapeDtypeStruct((B, 1, _K_V), jnp.int32),
            jax.ShapeDtypeStruct((B, 1, _N_REL), jnp.float32),
            jax.ShapeDtypeStruct((B, 1, _N_VAL), jnp.float32),
        ],
        scratch_shapes=[pltpu.VMEM((1, _N_PAD), jnp.float32),
                        pltpu.VMEM((_N_PAD, _D_SPACE), jnp.float32)],
        compiler_params=pltpu.CompilerParams(
            dimension_semantics=("parallel", "arbitrary")),
    )(x, imp3, W_proj, bp, neuron_emb, neuron_emb)

    return (iqk.reshape(B, _K_QK), iv.reshape(B, _K_V),
            rw.reshape(B, _N_REL), rw.reshape(B, _N_REL), vw.reshape(B, _N_VAL))


# speedup vs baseline: 1.0835x; 1.0835x over previous
import jax
import jax.numpy as jnp
from jax import lax
from jax.experimental import pallas as pl
from jax.experimental.pallas import tpu as pltpu

_D_SPACE = 64
_N_QK, _N_V, _N_REL, _N_VAL = 256, 128, 64, 32
_K_QK, _K_V, _K_REL, _K_VAL = 64, 32, 16, 3
_ST = 1024


_N_PAD = 640
_REGIONS = ((0, 256, 0.0), (256, 128, 0.0), (384, 128, 64.0), (512, 128, 96.0))


def _fused_kernel(x_ref, imp_ref, wp_ref, bp_ref, ea_ref, eb_ref,
                  iqk_ref, iv_ref, rw_ref, vw_ref, acc_ref):
    s = pl.program_id(1)
    ns = pl.num_programs(1)
    x = x_ref[0]
    h = jnp.dot(x, wp_ref[...], preferred_element_type=jnp.float32) + bp_ref[...]
    imp = imp_ref[0]
    e = jnp.concatenate([ea_ref[...], jnp.zeros((64, _D_SPACE), jnp.float32),
                         eb_ref[...], jnp.zeros((96, _D_SPACE), jnp.float32)],
                        axis=0)
    en = e / (jnp.sqrt(jnp.sum(e * e, axis=1, keepdims=True)) + 1e-12)
    lg = lax.dot_general(h, en, (((1,), (1,)), ((), ())),
                         preferred_element_type=jnp.float32)
    ex = jnp.exp(lg)
    st = ex.shape[0]
    invs = []
    for lo, width, padn in _REGIONS:
        d = jnp.sum(ex[:, lo:lo + width], axis=1, keepdims=True) - padn
        invs.append(jnp.broadcast_to(1.0 / d, (st, width)))
    p = ex * jnp.concatenate(invs, axis=1)
    contrib = jnp.dot(imp, p, preferred_element_type=jnp.float32)

    @pl.when(s == 0)
    def _():
        acc_ref[...] = contrib

    @pl.when(s != 0)
    def _():
        acc_ref[...] += contrib

    @pl.when(s == ns - 1)
    def _():
        acc = acc_ref[...]
        for lo, n, k, i_ref in ((0, _N_QK, _K_QK, iqk_ref),
                                (256, _N_V, _K_V, iv_ref)):
            w_row = acc[:, lo:lo + n]
            sel = _row_sel(w_row, k).astype(jnp.float32)
            i_ref[0] = _row_sorted_idx(sel, k)
        for lo, n, k, o_ref in ((384, _N_REL, _K_REL, rw_ref),
                                (512, _N_VAL, _K_VAL, vw_ref)):
            w_row = acc[:, lo:lo + n]
            sw = w_row * _row_sel(w_row, k).astype(jnp.float32)
            o_ref[0] = sw / (jnp.sum(sw, axis=1, keepdims=True) + 1e-8)


def _col_bcast(row, m):
    ones = jnp.ones((1, m), jnp.float32)
    return lax.dot_general(row, ones, (((0,), (0,)), ((), ())),
                           precision=lax.Precision.HIGHEST,
                           preferred_element_type=jnp.float32)


def _row_sel(w_row, k):
    n = w_row.shape[1]
    wj = _col_bcast(w_row, n)
    wi = jnp.broadcast_to(w_row, (n, n))
    ij = lax.broadcasted_iota(jnp.int32, (n, n), 0)
    ii = lax.broadcasted_iota(jnp.int32, (n, n), 1)
    beats = (wj > wi) | ((wj == wi) & (ij < ii))
    rank = jnp.sum(beats.astype(jnp.float32), axis=0, keepdims=True)
    return rank < float(k)


def _row_sorted_idx(self, k):
    n = self.shape[1]
    ij = lax.broadcasted_iota(jnp.int32, (n, n), 0)
    ii = lax.broadcasted_iota(jnp.int32, (n, n), 1)
    tri = (ij <= ii).astype(jnp.float32)
    csum = jnp.dot(self, tri, precision=lax.Precision.HIGHEST,
                   preferred_element_type=jnp.float32)
    pos = csum - 1.0
    pos_mat = _col_bcast(pos, k)
    sel_mat = _col_bcast(self, k)
    kio = lax.broadcasted_iota(jnp.int32, (n, k), 1).astype(jnp.float32)
    iio = lax.broadcasted_iota(jnp.int32, (n, k), 0).astype(jnp.float32)
    onehot = sel_mat * (pos_mat == kio).astype(jnp.float32)
    idx = jnp.sum(onehot * iio, axis=0, keepdims=True)
    return idx.astype(jnp.int32)


def kernel(x, importance, W_proj, b_proj, neuron_emb):
    B, S, D = x.shape
    bp = b_proj.reshape(1, _D_SPACE)
    ns = S // _ST

    osizes = (_K_QK, _K_V, _N_REL, _N_VAL)
    imp3 = importance.reshape(B, 1, S)
    iqk, iv, rw, vw = pl.pallas_call(
        _fused_kernel,
        grid=(B, ns),
        in_specs=[
            pl.BlockSpec((1, _ST, D), lambda b, s: (b, s, 0)),
            pl.BlockSpec((1, 1, _ST), lambda b, s: (b, 0, s)),
            pl.BlockSpec((D, _D_SPACE), lambda b, s: (0, 0)),
            pl.BlockSpec((1, _D_SPACE), lambda b, s: (0, 0)),
            pl.BlockSpec((448, _D_SPACE), lambda b, s: (0, 0)),
            pl.BlockSpec((32, _D_SPACE), lambda b, s: (14, 0)),
        ],
        out_specs=[pl.BlockSpec((1, 1, n), lambda b, s: (b, 0, 0)) for n in osizes],
        out_shape=[
            jax.ShapeDtypeStruct((B, 1, _K_QK), jnp.int32),
            jax.ShapeDtypeStruct((B, 1, _K_V), jnp.int32),
            jax.ShapeDtypeStruct((B, 1, _N_REL), jnp.float32),
            jax.ShapeDtypeStruct((B, 1, _N_VAL), jnp.float32),
        ],
        scratch_shapes=[pltpu.VMEM((1, _N_PAD), jnp.float32)],
        compiler_params=pltpu.CompilerParams(
            dimension_semantics=("parallel", "arbitrary")),
    )(x, imp3, W_proj, bp, neuron_emb, neuron_emb)

    return (iqk.reshape(B, _K_QK), iv.reshape(B, _K_V),
            rw.reshape(B, _N_REL), rw.reshape(B, _N_REL), vw.reshape(B, _N_VAL))
